# baseline (speedup 1.0000x reference)
import functools

import jax
import jax.numpy as jnp
from jax import lax
from jax.experimental import pallas as pl
from jax.experimental.pallas import tpu as pltpu
from jax.experimental.pallas import tpu_sc as plsc

VOCAB = 1000000
BATCH = 16384
UNITS = 10

_NC = 2
_NS = 16
_NW = _NC * _NS
_BPW = BATCH // _NW
_ICH = 128
_KCH = _BPW // _ICH

_BLK = 2048
_G = BATCH // _BLK

_mesh = plsc.VectorSubcoreMesh(
    core_axis_name="c", subcore_axis_name="s", num_cores=_NC, num_subcores=_NS
)


@functools.partial(
    pl.kernel,
    out_type=jax.ShapeDtypeStruct((_NW, _BPW), jnp.int32),
    mesh=_mesh,
    scratch_types=[
        pltpu.VMEM((_KCH, _ICH), jnp.int32),
        pltpu.VMEM((_BPW,), jnp.int32),
        pltpu.SemaphoreType.DMA,
    ],
)
def _gather_sc(table_h, idx_h, out_h, idx_v, ids_v, sem):
    wid = lax.axis_index("s") * _NC + lax.axis_index("c")
    pltpu.sync_copy(idx_h.at[wid], idx_v)
    copies = [
        pltpu.async_copy(
            table_h.at[idx_v.at[k]], ids_v.at[pl.ds(k * _ICH, _ICH)], sem
        )
        for k in range(_KCH)
    ]
    for c in copies:
        c.wait()
    pltpu.sync_copy(ids_v, out_h.at[wid])


def _affine_tc(ids_ref, w_ref, b_ref, o_ref):
    ids = ids_ref[...].astype(jnp.float32)
    o_ref[...] = ids * w_ref[...] + b_ref[...]


def kernel(inputs, lookup_table, W, b):
    idx = inputs.reshape(-1).astype(jnp.int32).reshape(_NW, _KCH, _ICH)
    ids = _gather_sc(lookup_table, idx).reshape(BATCH, 1)
    out = pl.pallas_call(
        _affine_tc,
        out_shape=jax.ShapeDtypeStruct((BATCH, UNITS), jnp.float32),
        grid=(_G,),
        in_specs=[
            pl.BlockSpec((_BLK, 1), lambda i: (i, 0)),
            pl.BlockSpec((1, UNITS), lambda i: (0, 0)),
            pl.BlockSpec((1, UNITS), lambda i: (0, 0)),
        ],
        out_specs=pl.BlockSpec((_BLK, UNITS), lambda i: (i, 0)),
    )(ids, W.astype(jnp.float32), b.reshape(1, UNITS).astype(jnp.float32))
    return out


# speedup vs baseline: 1.7304x; 1.7304x over previous
import functools

import jax
import jax.numpy as jnp
from jax import lax
from jax.experimental import pallas as pl
from jax.experimental.pallas import tpu as pltpu
from jax.experimental.pallas import tpu_sc as plsc

VOCAB = 1000000
BATCH = 16384
UNITS = 10

_NC = 2
_NS = 16
_NW = _NC * _NS
_BPW = BATCH // _NW
_ICH = 128
_KCH = _BPW // _ICH
_LANES = 16

_mesh = plsc.VectorSubcoreMesh(
    core_axis_name="c", subcore_axis_name="s", num_cores=_NC, num_subcores=_NS
)


@functools.partial(
    pl.kernel,
    out_type=jax.ShapeDtypeStruct((_NW, _KCH, UNITS, _ICH), jnp.float32),
    mesh=_mesh,
    scratch_types=[
        pltpu.VMEM((_KCH, _ICH), jnp.int32),
        pltpu.VMEM((_BPW,), jnp.int32),
        pltpu.VMEM((2, _LANES), jnp.float32),
        pltpu.VMEM((_KCH, UNITS, _ICH), jnp.float32),
        pltpu.SemaphoreType.DMA,
        pltpu.SemaphoreType.DMA,
        pltpu.SemaphoreType.DMA,
        pltpu.SemaphoreType.DMA,
        pltpu.SemaphoreType.DMA,
        pltpu.SemaphoreType.DMA,
    ],
)
def _lookup_affine(
    table_h, idx_h, wb_h, out_h, idx_v, ids_v, wb_v, out_v, wb_sem, *sems
):
    gsems, osem = sems[:_KCH], sems[_KCH]
    wid = lax.axis_index("s") * _NC + lax.axis_index("c")
    wcp = pltpu.async_copy(wb_h, wb_v, wb_sem)
    pltpu.sync_copy(idx_h.at[wid], idx_v)
    copies = [
        pltpu.async_copy(
            table_h.at[idx_v.at[k]], ids_v.at[pl.ds(k * _ICH, _ICH)], gsems[k]
        )
        for k in range(_KCH)
    ]
    wcp.wait()
    wrow = wb_v[0]
    brow = wb_v[1]
    ws = [wrow[j] for j in range(UNITS)]
    bs = [brow[j] for j in range(UNITS)]
    ocs = []
    for k in range(_KCH):
        copies[k].wait()
        for i in range(_ICH // _LANES):
            v = ids_v[pl.ds(k * _ICH + i * _LANES, _LANES)].astype(jnp.float32)
            for j in range(UNITS):
                out_v[k, j, pl.ds(i * _LANES, _LANES)] = v * ws[j] + bs[j]
        ocs.append(pltpu.async_copy(out_v.at[k], out_h.at[wid, k], osem))
    for oc in ocs:
        oc.wait()


def kernel(inputs, lookup_table, W, b):
    idx = inputs.reshape(-1).astype(jnp.int32).reshape(_NW, _KCH, _ICH)
    wb = jnp.zeros((2, _LANES), jnp.float32)
    wb = wb.at[0, :UNITS].set(W[0].astype(jnp.float32))
    wb = wb.at[1, :UNITS].set(b.astype(jnp.float32))
    out = _lookup_affine(lookup_table, idx, wb)
    return out.transpose(0, 1, 3, 2).reshape(BATCH, UNITS)


# speedup vs baseline: 1.7367x; 1.0036x over previous
import functools

import jax
import jax.numpy as jnp
from jax import lax
from jax.experimental import pallas as pl
from jax.experimental.pallas import tpu as pltpu
from jax.experimental.pallas import tpu_sc as plsc

VOCAB = 1000000
BATCH = 16384
UNITS = 10

_NC = 2
_NS = 16
_NW = _NC * _NS
_BPW = BATCH // _NW
_ICH = 128
_KCH = _BPW // _ICH
_LANES = 16

_mesh = plsc.VectorSubcoreMesh(
    core_axis_name="c", subcore_axis_name="s", num_cores=_NC, num_subcores=_NS
)


@functools.partial(
    pl.kernel,
    out_type=jax.ShapeDtypeStruct((_NW, _KCH, UNITS, _ICH), jnp.float32),
    mesh=_mesh,
    scratch_types=[
        pltpu.VMEM((_KCH, _ICH), jnp.int32),
        pltpu.VMEM((_BPW,), jnp.int32),
        pltpu.VMEM((2, _LANES), jnp.float32),
        pltpu.VMEM((_KCH, UNITS, _ICH), jnp.float32),
        pltpu.SemaphoreType.DMA,
        pltpu.SemaphoreType.DMA,
        pltpu.SemaphoreType.DMA,
        pltpu.SemaphoreType.DMA,
        pltpu.SemaphoreType.DMA,
        pltpu.SemaphoreType.DMA,
        pltpu.SemaphoreType.DMA,
        pltpu.SemaphoreType.DMA,
        pltpu.SemaphoreType.DMA,
        pltpu.SemaphoreType.DMA,
    ],
)
def _lookup_affine(
    table_h, idx_h, wb_h, out_h, idx_v, ids_v, wb_v, out_v, wb_sem, *sems
):
    isems, gsems, osem = sems[:_KCH], sems[_KCH:2 * _KCH], sems[2 * _KCH]
    wid = lax.axis_index("s") * _NC + lax.axis_index("c")
    wcp = pltpu.async_copy(wb_h, wb_v, wb_sem)
    icps = [
        pltpu.async_copy(idx_h.at[wid, k], idx_v.at[k], isems[k])
        for k in range(_KCH)
    ]
    copies = []
    for k in range(_KCH):
        icps[k].wait()
        copies.append(
            pltpu.async_copy(
                table_h.at[idx_v.at[k]], ids_v.at[pl.ds(k * _ICH, _ICH)],
                gsems[k]
            )
        )
    wcp.wait()
    wrow = wb_v[0]
    brow = wb_v[1]
    ws = [wrow[j] for j in range(UNITS)]
    bs = [brow[j] for j in range(UNITS)]
    ocs = []
    for k in range(_KCH):
        copies[k].wait()
        for i in range(_ICH // _LANES):
            v = ids_v[pl.ds(k * _ICH + i * _LANES, _LANES)].astype(jnp.float32)
            for j in range(UNITS):
                out_v[k, j, pl.ds(i * _LANES, _LANES)] = v * ws[j] + bs[j]
        ocs.append(pltpu.async_copy(out_v.at[k], out_h.at[wid, k], osem))
    for oc in ocs:
        oc.wait()


def kernel(inputs, lookup_table, W, b):
    idx = inputs.reshape(-1).astype(jnp.int32).reshape(_NW, _KCH, _ICH)
    wb = jnp.zeros((2, _LANES), jnp.float32)
    wb = wb.at[0, :UNITS].set(W[0].astype(jnp.float32))
    wb = wb.at[1, :UNITS].set(b.astype(jnp.float32))
    out = _lookup_affine(lookup_table, idx, wb)
    return out.transpose(0, 1, 3, 2).reshape(BATCH, UNITS)
